# baseline (speedup 1.0000x reference)
import functools

import jax
import jax.numpy as jnp
from jax import lax
from jax.experimental import pallas as pl
from jax.experimental.pallas import tpu as pltpu
from jax.experimental.pallas import tpu_sc as plsc

NUM_EMB = 1_000_000
DIM = 64
BATCH = 16384
NS, L = 16, 16
NW = NS
RANGE = 65536
POS_PAD = RANGE * NW
B_PER_W = BATCH // NW
CHUNK = 128
NCHUNK = B_PER_W // CHUNK
ROWS_BUF = 512
NIT = BATCH // L

_mesh = plsc.VectorSubcoreMesh(
    core_axis_name="c", subcore_axis_name="s", num_cores=1)


@functools.partial(
    pl.kernel,
    mesh=_mesh,
    compiler_params=pltpu.CompilerParams(needs_layout_passes=False),
    out_type=jax.ShapeDtypeStruct((POS_PAD,), jnp.int32),
    scratch_types=[
        pltpu.VMEM((BATCH,), jnp.int32),
        pltpu.VMEM((RANGE,), jnp.int32),
    ],
)
def _build_pos(nid_hbm, pos_hbm, nid_v, slab_v):
    wid = lax.axis_index("s")
    pltpu.sync_copy(nid_hbm, nid_v)
    lanes = lax.iota(jnp.int32, L)

    def emit(jvec, v, last):
        m = last & (lax.shift_right_arithmetic(v, 16) == wid)
        plsc.store_scatter(slab_v, [v & (RANGE - 1)], jvec, mask=m)

    def body(it, carry):
        v, last, jvec = carry
        v_n = nid_v[pl.ds((it + 1) * L, L)]
        last_n = plsc.scan_count(v_n)[1]
        emit(jvec, v, last)
        return v_n, last_n, jvec + L

    v0 = nid_v[pl.ds(0, L)]
    v_f, last_f, j_f = lax.fori_loop(
        0, NIT - 1, body, (v0, plsc.scan_count(v0)[1], lanes), unroll=16)
    emit(j_f, v_f, last_f)
    pltpu.sync_copy(slab_v, pos_hbm.at[pl.ds(wid * RANGE, RANGE)])


HALF = BATCH // 2
H_PER_W = HALF // NW
H_NCHUNK = H_PER_W // CHUNK


def _make_gather(half):
    @functools.partial(
        pl.kernel,
        mesh=_mesh,
        compiler_params=pltpu.CompilerParams(
            needs_layout_passes=False, use_tc_tiling_on_sc=False),
        out_type=jax.ShapeDtypeStruct((HALF, DIM), jnp.float32),
        scratch_types=[
            pltpu.VMEM((H_NCHUNK, CHUNK), jnp.int32),
            pltpu.VMEM((H_PER_W,), jnp.int32),
            pltpu.VMEM((H_PER_W, DIM), jnp.float32),
            pltpu.SemaphoreType.DMA,
            pltpu.SemaphoreType.DMA,
        ],
    )
    def _gather(nid_hbm, pos_hbm, x_hbm, out_hbm, nid_v, w_v, rows_v, s1, s2):
        wid = lax.axis_index("s")
        base = wid * H_PER_W
        pltpu.sync_copy(
            nid_hbm.at[pl.ds((half * HALF + wid * H_PER_W) // CHUNK,
                             H_NCHUNK)], nid_v)
        h1 = [pltpu.async_copy(pos_hbm.at[nid_v.at[c]],
                               w_v.at[pl.ds(c * CHUNK, CHUNK)], s1)
              for c in range(H_NCHUNK)]
        h2 = []
        for c in range(H_NCHUNK):
            h1[c].wait()
            h2.append(pltpu.async_copy(
                x_hbm.at[w_v.at[pl.ds(c * CHUNK, CHUNK)]],
                rows_v.at[pl.ds(c * CHUNK, CHUNK)], s2))
        for h in h2:
            h.wait()
        pltpu.sync_copy(rows_v, out_hbm.at[pl.ds(base, H_PER_W)])

    return _gather


_gather_a = _make_gather(0)
_gather_b = _make_gather(1)


def kernel(emb, x, n_id):
    del emb
    pos = _build_pos(n_id)
    nid2 = n_id.reshape(NW * NCHUNK, CHUNK)
    return jnp.concatenate(
        [_gather_a(nid2, pos, x), _gather_b(nid2, pos, x)], axis=0)


# speedup vs baseline: 1.1811x; 1.1811x over previous
import functools

import jax
import jax.numpy as jnp
from jax import lax
from jax.experimental import pallas as pl
from jax.experimental.pallas import tpu as pltpu
from jax.experimental.pallas import tpu_sc as plsc

NUM_EMB = 1_000_000
DIM = 64
BATCH = 16384
NS, L = 16, 16
NW = NS
RANGE = 65536
POS_PAD = RANGE * NW
B_PER_W = BATCH // NW
CHUNK = 128
NCHUNK = B_PER_W // CHUNK
ROWS_BUF = 512
NIT = BATCH // L

_mesh = plsc.VectorSubcoreMesh(
    core_axis_name="c", subcore_axis_name="s", num_cores=1)


@functools.partial(
    pl.kernel,
    mesh=_mesh,
    compiler_params=pltpu.CompilerParams(needs_layout_passes=False),
    out_type=jax.ShapeDtypeStruct((POS_PAD,), jnp.int32),
    scratch_types=[
        pltpu.VMEM((BATCH,), jnp.int32),
        pltpu.VMEM((RANGE,), jnp.int32),
    ],
)
def _build_pos(nid_hbm, pos_hbm, nid_v, slab_v):
    wid = lax.axis_index("s")
    pltpu.sync_copy(nid_hbm, nid_v)
    lanes = lax.iota(jnp.int32, L)

    def emit(jvec, v, last):
        m = last & (lax.shift_right_arithmetic(v, 16) == wid)
        plsc.store_scatter(slab_v, [v & (RANGE - 1)], jvec, mask=m)

    def body(it, carry):
        v, last, jvec = carry
        v_n = nid_v[pl.ds((it + 1) * L, L)]
        last_n = plsc.scan_count(v_n)[1]
        emit(jvec, v, last)
        return v_n, last_n, jvec + L

    v0 = nid_v[pl.ds(0, L)]
    v_f, last_f, j_f = lax.fori_loop(
        0, NIT - 1, body, (v0, plsc.scan_count(v0)[1], lanes), unroll=16)
    emit(j_f, v_f, last_f)
    pltpu.sync_copy(slab_v, pos_hbm.at[pl.ds(wid * RANGE, RANGE)])


@functools.partial(
    pl.kernel,
    mesh=_mesh,
    compiler_params=pltpu.CompilerParams(
        needs_layout_passes=False, use_tc_tiling_on_sc=False),
    out_type=jax.ShapeDtypeStruct((BATCH, DIM), jnp.float32),
    scratch_types=[
        pltpu.VMEM((NCHUNK, CHUNK), jnp.int32),
        pltpu.VMEM((B_PER_W,), jnp.int32),
        pltpu.VMEM((B_PER_W, DIM), jnp.float32),
        pltpu.SemaphoreType.DMA,
        pltpu.SemaphoreType.DMA,
    ],
)
def _gather_out(nid_hbm, pos_hbm, x_hbm, out_hbm, nid_v, w_v, rows_v, s1, s2):
    wid = lax.axis_index("s")
    base = wid * B_PER_W
    pltpu.sync_copy(nid_hbm.at[pl.ds(wid * NCHUNK, NCHUNK)], nid_v)
    h1 = [pltpu.async_copy(pos_hbm.at[nid_v.at[c]],
                           w_v.at[pl.ds(c * CHUNK, CHUNK)], s1)
          for c in range(NCHUNK)]
    h2 = []
    for c in range(NCHUNK):
        h1[c].wait()
        h2.append(pltpu.async_copy(
            x_hbm.at[w_v.at[pl.ds(c * CHUNK, CHUNK)]],
            rows_v.at[pl.ds(c * CHUNK, CHUNK)], s2))
    h3 = []
    for c in range(NCHUNK):
        h2[c].wait()
        h3.append(pltpu.async_copy(
            rows_v.at[pl.ds(c * CHUNK, CHUNK)],
            out_hbm.at[pl.ds(base + c * CHUNK, CHUNK)], s1))
    for h in h3:
        h.wait()


def kernel(emb, x, n_id):
    del emb
    pos = _build_pos(n_id)
    nid2 = n_id.reshape(NW * NCHUNK, CHUNK)
    return _gather_out(nid2, pos, x)


# speedup vs baseline: 1.2259x; 1.0380x over previous
import functools

import jax
import jax.numpy as jnp
from jax import lax
from jax.experimental import pallas as pl
from jax.experimental.pallas import tpu as pltpu
from jax.experimental.pallas import tpu_sc as plsc

NUM_EMB = 1_000_000
DIM = 64
BATCH = 16384
NS, L = 16, 16
NW = NS
RANGE = 65536
POS_PAD = RANGE * NW
B_PER_W = BATCH // NW
CHUNK = 128
NCHUNK = B_PER_W // CHUNK
ROWS_BUF = 512
NIT = BATCH // L

_mesh = plsc.VectorSubcoreMesh(
    core_axis_name="c", subcore_axis_name="s", num_cores=1)


@functools.partial(
    pl.kernel,
    mesh=_mesh,
    compiler_params=pltpu.CompilerParams(needs_layout_passes=False),
    out_type=jax.ShapeDtypeStruct((POS_PAD,), jnp.int32),
    scratch_types=[
        pltpu.VMEM((BATCH,), jnp.int32),
        pltpu.VMEM((RANGE,), jnp.int32),
    ],
)
def _build_pos(nid_hbm, pos_hbm, nid_v, slab_v):
    wid = lax.axis_index("s")
    pltpu.sync_copy(nid_hbm, nid_v)
    lanes = lax.iota(jnp.int32, L)

    def emit(jvec, v, last):
        m = last & (lax.shift_right_arithmetic(v, 16) == wid)
        plsc.store_scatter(slab_v, [v & (RANGE - 1)], jvec, mask=m)

    def body(it, carry):
        v, last, v1, last1, jvec = carry
        v_n = nid_v[pl.ds((it + 2) * L, L)]
        last_n = plsc.scan_count(v_n)[1]
        emit(jvec, v, last)
        return v1, last1, v_n, last_n, jvec + L

    v0 = nid_v[pl.ds(0, L)]
    v1 = nid_v[pl.ds(L, L)]
    va, la, vb, lb, j_f = lax.fori_loop(
        0, NIT - 2, body,
        (v0, plsc.scan_count(v0)[1], v1, plsc.scan_count(v1)[1], lanes),
        unroll=16)
    emit(j_f, va, la)
    emit(j_f + L, vb, lb)
    pltpu.sync_copy(slab_v, pos_hbm.at[pl.ds(wid * RANGE, RANGE)])


@functools.partial(
    pl.kernel,
    mesh=_mesh,
    compiler_params=pltpu.CompilerParams(
        needs_layout_passes=False, use_tc_tiling_on_sc=False),
    out_type=jax.ShapeDtypeStruct((BATCH, DIM), jnp.float32),
    scratch_types=[
        pltpu.VMEM((NCHUNK, CHUNK), jnp.int32),
        pltpu.VMEM((B_PER_W,), jnp.int32),
        pltpu.VMEM((B_PER_W, DIM), jnp.float32),
        pltpu.SemaphoreType.DMA,
        pltpu.SemaphoreType.DMA,
    ],
)
def _gather_out(nid_hbm, pos_hbm, x_hbm, out_hbm, nid_v, w_v, rows_v, s1, s2):
    wid = lax.axis_index("s")
    base = wid * B_PER_W
    pltpu.sync_copy(nid_hbm.at[pl.ds(wid * NCHUNK, NCHUNK)], nid_v)
    h1 = [pltpu.async_copy(pos_hbm.at[nid_v.at[c]],
                           w_v.at[pl.ds(c * CHUNK, CHUNK)], s1)
          for c in range(NCHUNK)]
    h2 = []
    for c in range(NCHUNK):
        h1[c].wait()
        h2.append(pltpu.async_copy(
            x_hbm.at[w_v.at[pl.ds(c * CHUNK, CHUNK)]],
            rows_v.at[pl.ds(c * CHUNK, CHUNK)], s2))
    h3 = []
    for c in range(NCHUNK):
        h2[c].wait()
        h3.append(pltpu.async_copy(
            rows_v.at[pl.ds(c * CHUNK, CHUNK)],
            out_hbm.at[pl.ds(base + c * CHUNK, CHUNK)], s1))
    for h in h3:
        h.wait()


def kernel(emb, x, n_id):
    del emb
    pos = _build_pos(n_id)
    nid2 = n_id.reshape(NW * NCHUNK, CHUNK)
    return _gather_out(nid2, pos, x)


# speedup vs baseline: 1.2576x; 1.0258x over previous
import functools

import jax
import jax.numpy as jnp
from jax import lax
from jax.experimental import pallas as pl
from jax.experimental.pallas import tpu as pltpu
from jax.experimental.pallas import tpu_sc as plsc

NUM_EMB = 1_000_000
DIM = 64
BATCH = 16384
NS, L = 16, 16
NW = NS
RANGE = 65536
POS_PAD = RANGE * NW
B_PER_W = BATCH // NW
CHUNK = 128
NCHUNK = B_PER_W // CHUNK
ROWS_BUF = 512
NIT = BATCH // L

_mesh = plsc.VectorSubcoreMesh(
    core_axis_name="c", subcore_axis_name="s", num_cores=1)


@functools.partial(
    pl.kernel,
    mesh=_mesh,
    compiler_params=pltpu.CompilerParams(needs_layout_passes=False),
    out_type=jax.ShapeDtypeStruct((POS_PAD,), jnp.int32),
    scratch_types=[
        pltpu.VMEM((BATCH,), jnp.int32),
        pltpu.VMEM((RANGE,), jnp.int32),
    ],
)
def _build_pos(nid_hbm, pos_hbm, nid_v, slab_v):
    wid = lax.axis_index("s")
    pltpu.sync_copy(nid_hbm, nid_v)
    lanes = lax.iota(jnp.int32, L)

    def emit(jvec, v, last):
        m = last & (lax.shift_right_arithmetic(v, 16) == wid)
        plsc.store_scatter(slab_v, [v & (RANGE - 1)], jvec, mask=m)

    def body(it, carry):
        v, last, v1, last1, v2, last2, jvec = carry
        v_n = nid_v[pl.ds((it + 3) * L, L)]
        last_n = plsc.scan_count(v_n)[1]
        emit(jvec, v, last)
        return v1, last1, v2, last2, v_n, last_n, jvec + L

    v0 = nid_v[pl.ds(0, L)]
    v1 = nid_v[pl.ds(L, L)]
    v2 = nid_v[pl.ds(2 * L, L)]
    va, la, vb, lb, vc, lc, j_f = lax.fori_loop(
        0, NIT - 3, body,
        (v0, plsc.scan_count(v0)[1], v1, plsc.scan_count(v1)[1],
         v2, plsc.scan_count(v2)[1], lanes),
        unroll=16)
    emit(j_f, va, la)
    emit(j_f + L, vb, lb)
    emit(j_f + 2 * L, vc, lc)
    pltpu.sync_copy(slab_v, pos_hbm.at[pl.ds(wid * RANGE, RANGE)])


@functools.partial(
    pl.kernel,
    mesh=_mesh,
    compiler_params=pltpu.CompilerParams(
        needs_layout_passes=False, use_tc_tiling_on_sc=False),
    out_type=jax.ShapeDtypeStruct((BATCH, DIM), jnp.float32),
    scratch_types=[
        pltpu.VMEM((NCHUNK, CHUNK), jnp.int32),
        pltpu.VMEM((B_PER_W,), jnp.int32),
        pltpu.VMEM((B_PER_W, DIM), jnp.float32),
        pltpu.SemaphoreType.DMA,
        pltpu.SemaphoreType.DMA,
    ],
)
def _gather_out(nid_hbm, pos_hbm, x_hbm, out_hbm, nid_v, w_v, rows_v, s1, s2):
    wid = lax.axis_index("s")
    base = wid * B_PER_W
    pltpu.sync_copy(nid_hbm.at[pl.ds(wid * NCHUNK, NCHUNK)], nid_v)
    h1 = [pltpu.async_copy(pos_hbm.at[nid_v.at[c]],
                           w_v.at[pl.ds(c * CHUNK, CHUNK)], s1)
          for c in range(NCHUNK)]
    h2 = []
    for c in range(NCHUNK):
        h1[c].wait()
        h2.append(pltpu.async_copy(
            x_hbm.at[w_v.at[pl.ds(c * CHUNK, CHUNK)]],
            rows_v.at[pl.ds(c * CHUNK, CHUNK)], s2))
    h3 = []
    for c in range(NCHUNK):
        h2[c].wait()
        h3.append(pltpu.async_copy(
            rows_v.at[pl.ds(c * CHUNK, CHUNK)],
            out_hbm.at[pl.ds(base + c * CHUNK, CHUNK)], s1))
    for h in h3:
        h.wait()


def kernel(emb, x, n_id):
    del emb
    pos = _build_pos(n_id)
    nid2 = n_id.reshape(NW * NCHUNK, CHUNK)
    return _gather_out(nid2, pos, x)


# speedup vs baseline: 1.2809x; 1.0185x over previous
import functools

import jax
import jax.numpy as jnp
from jax import lax
from jax.experimental import pallas as pl
from jax.experimental.pallas import tpu as pltpu
from jax.experimental.pallas import tpu_sc as plsc

NUM_EMB = 1_000_000
DIM = 64
BATCH = 16384
NS, L = 16, 16
NW = NS
RANGE = 65536
POS_PAD = RANGE * NW
B_PER_W = BATCH // NW
CHUNK = 128
NCHUNK = B_PER_W // CHUNK
ROWS_BUF = 512
NIT = BATCH // L

_mesh = plsc.VectorSubcoreMesh(
    core_axis_name="c", subcore_axis_name="s", num_cores=1)


@functools.partial(
    pl.kernel,
    mesh=_mesh,
    compiler_params=pltpu.CompilerParams(needs_layout_passes=False),
    out_type=jax.ShapeDtypeStruct((POS_PAD,), jnp.int32),
    scratch_types=[
        pltpu.VMEM((BATCH,), jnp.int32),
        pltpu.VMEM((RANGE,), jnp.int32),
    ],
)
def _build_pos(nid_hbm, pos_hbm, nid_v, slab_v):
    wid = lax.axis_index("s")
    pltpu.sync_copy(nid_hbm, nid_v)
    lanes = lax.iota(jnp.int32, L)

    def emit(jvec, v, last):
        m = last & (lax.shift_right_arithmetic(v, 16) == wid)
        plsc.store_scatter(slab_v, [v & (RANGE - 1)], jvec, mask=m)

    DEPTH = 5

    def body(it, carry):
        jvec = carry[-1]
        pipe = carry[:-1]
        v_n = nid_v[pl.ds((it + DEPTH) * L, L)]
        last_n = plsc.scan_count(v_n)[1]
        emit(jvec, pipe[0], pipe[1])
        return pipe[2:] + (v_n, last_n, jvec + L)

    prime = []
    for k in range(DEPTH):
        vk = nid_v[pl.ds(k * L, L)]
        prime += [vk, plsc.scan_count(vk)[1]]
    fin = lax.fori_loop(0, NIT - DEPTH, body, (*prime, lanes), unroll=16)
    j_f = fin[-1]
    for k in range(DEPTH):
        emit(j_f + k * L, fin[2 * k], fin[2 * k + 1])
    pltpu.sync_copy(slab_v, pos_hbm.at[pl.ds(wid * RANGE, RANGE)])


@functools.partial(
    pl.kernel,
    mesh=_mesh,
    compiler_params=pltpu.CompilerParams(
        needs_layout_passes=False, use_tc_tiling_on_sc=False),
    out_type=jax.ShapeDtypeStruct((BATCH, DIM), jnp.float32),
    scratch_types=[
        pltpu.VMEM((NCHUNK, CHUNK), jnp.int32),
        pltpu.VMEM((B_PER_W,), jnp.int32),
        pltpu.VMEM((B_PER_W, DIM), jnp.float32),
        pltpu.SemaphoreType.DMA,
        pltpu.SemaphoreType.DMA,
    ],
)
def _gather_out(nid_hbm, pos_hbm, x_hbm, out_hbm, nid_v, w_v, rows_v, s1, s2):
    wid = lax.axis_index("s")
    base = wid * B_PER_W
    pltpu.sync_copy(nid_hbm.at[pl.ds(wid * NCHUNK, NCHUNK)], nid_v)
    h1 = [pltpu.async_copy(pos_hbm.at[nid_v.at[c]],
                           w_v.at[pl.ds(c * CHUNK, CHUNK)], s1)
          for c in range(NCHUNK)]
    h2 = []
    for c in range(NCHUNK):
        h1[c].wait()
        h2.append(pltpu.async_copy(
            x_hbm.at[w_v.at[pl.ds(c * CHUNK, CHUNK)]],
            rows_v.at[pl.ds(c * CHUNK, CHUNK)], s2))
    h3 = []
    for c in range(NCHUNK):
        h2[c].wait()
        h3.append(pltpu.async_copy(
            rows_v.at[pl.ds(c * CHUNK, CHUNK)],
            out_hbm.at[pl.ds(base + c * CHUNK, CHUNK)], s1))
    for h in h3:
        h.wait()


def kernel(emb, x, n_id):
    del emb
    pos = _build_pos(n_id)
    nid2 = n_id.reshape(NW * NCHUNK, CHUNK)
    return _gather_out(nid2, pos, x)


# speedup vs baseline: 1.2999x; 1.0148x over previous
import functools

import jax
import jax.numpy as jnp
from jax import lax
from jax.experimental import pallas as pl
from jax.experimental.pallas import tpu as pltpu
from jax.experimental.pallas import tpu_sc as plsc

NUM_EMB = 1_000_000
DIM = 64
BATCH = 16384
NS, L = 16, 16
NW = NS
RANGE = 65536
POS_PAD = RANGE * NW
B_PER_W = BATCH // NW
CHUNK = 128
NCHUNK = B_PER_W // CHUNK
ROWS_BUF = 512
NIT = BATCH // L

_mesh = plsc.VectorSubcoreMesh(
    core_axis_name="c", subcore_axis_name="s", num_cores=1)


@functools.partial(
    pl.kernel,
    mesh=_mesh,
    compiler_params=pltpu.CompilerParams(needs_layout_passes=False),
    out_type=jax.ShapeDtypeStruct((POS_PAD,), jnp.int32),
    scratch_types=[
        pltpu.VMEM((BATCH,), jnp.int32),
        pltpu.VMEM((RANGE,), jnp.int32),
    ],
)
def _build_pos(nid_hbm, pos_hbm, nid_v, slab_v):
    wid = lax.axis_index("s")
    pltpu.sync_copy(nid_hbm, nid_v)
    lanes = lax.iota(jnp.int32, L)

    def emit(jvec, v, last):
        m = last & (lax.shift_right_arithmetic(v, 16) == wid)
        plsc.store_scatter(slab_v, [v & (RANGE - 1)], jvec, mask=m)

    DEPTH = 8

    def body(it, carry):
        jvec = carry[-1]
        pipe = carry[:-1]
        v_n = nid_v[pl.ds((it + DEPTH) * L, L)]
        last_n = plsc.scan_count(v_n)[1]
        emit(jvec, pipe[0], pipe[1])
        return pipe[2:] + (v_n, last_n, jvec + L)

    prime = []
    for k in range(DEPTH):
        vk = nid_v[pl.ds(k * L, L)]
        prime += [vk, plsc.scan_count(vk)[1]]
    fin = lax.fori_loop(0, NIT - DEPTH, body, (*prime, lanes), unroll=16)
    j_f = fin[-1]
    for k in range(DEPTH):
        emit(j_f + k * L, fin[2 * k], fin[2 * k + 1])
    pltpu.sync_copy(slab_v, pos_hbm.at[pl.ds(wid * RANGE, RANGE)])


@functools.partial(
    pl.kernel,
    mesh=_mesh,
    compiler_params=pltpu.CompilerParams(
        needs_layout_passes=False, use_tc_tiling_on_sc=False),
    out_type=jax.ShapeDtypeStruct((BATCH, DIM), jnp.float32),
    scratch_types=[
        pltpu.VMEM((NCHUNK, CHUNK), jnp.int32),
        pltpu.VMEM((B_PER_W,), jnp.int32),
        pltpu.VMEM((B_PER_W, DIM), jnp.float32),
        pltpu.SemaphoreType.DMA,
        pltpu.SemaphoreType.DMA,
    ],
)
def _gather_out(nid_hbm, pos_hbm, x_hbm, out_hbm, nid_v, w_v, rows_v, s1, s2):
    wid = lax.axis_index("s")
    base = wid * B_PER_W
    pltpu.sync_copy(nid_hbm.at[pl.ds(wid * NCHUNK, NCHUNK)], nid_v)
    h1 = [pltpu.async_copy(pos_hbm.at[nid_v.at[c]],
                           w_v.at[pl.ds(c * CHUNK, CHUNK)], s1)
          for c in range(NCHUNK)]
    h2 = []
    for c in range(NCHUNK):
        h1[c].wait()
        h2.append(pltpu.async_copy(
            x_hbm.at[w_v.at[pl.ds(c * CHUNK, CHUNK)]],
            rows_v.at[pl.ds(c * CHUNK, CHUNK)], s2))
    h3 = []
    for c in range(NCHUNK):
        h2[c].wait()
        h3.append(pltpu.async_copy(
            rows_v.at[pl.ds(c * CHUNK, CHUNK)],
            out_hbm.at[pl.ds(base + c * CHUNK, CHUNK)], s1))
    for h in h3:
        h.wait()


def kernel(emb, x, n_id):
    del emb
    pos = _build_pos(n_id)
    nid2 = n_id.reshape(NW * NCHUNK, CHUNK)
    return _gather_out(nid2, pos, x)


# speedup vs baseline: 1.3007x; 1.0007x over previous
import functools

import jax
import jax.numpy as jnp
from jax import lax
from jax.experimental import pallas as pl
from jax.experimental.pallas import tpu as pltpu
from jax.experimental.pallas import tpu_sc as plsc

NUM_EMB = 1_000_000
DIM = 64
BATCH = 16384
NS, L = 16, 16
NW = NS
RANGE = 65536
POS_PAD = RANGE * NW
B_PER_W = BATCH // NW
CHUNK = 128
NCHUNK = B_PER_W // CHUNK
ROWS_BUF = 512
NIT = BATCH // L

_mesh = plsc.VectorSubcoreMesh(
    core_axis_name="c", subcore_axis_name="s", num_cores=1)


@functools.partial(
    pl.kernel,
    mesh=_mesh,
    compiler_params=pltpu.CompilerParams(needs_layout_passes=False),
    out_type=jax.ShapeDtypeStruct((POS_PAD,), jnp.int32),
    scratch_types=[
        pltpu.VMEM((BATCH,), jnp.int32),
        pltpu.VMEM((RANGE,), jnp.int32),
    ],
)
def _build_pos(nid_hbm, pos_hbm, nid_v, slab_v):
    wid = lax.axis_index("s")
    pltpu.sync_copy(nid_hbm, nid_v)
    lanes = lax.iota(jnp.int32, L)

    def emit(jvec, v, last):
        m = last & (lax.shift_right_arithmetic(v, 16) == wid)
        plsc.store_scatter(slab_v, [v & (RANGE - 1)], jvec, mask=m)

    DEPTH = 12

    def body(it, carry):
        jvec = carry[-1]
        pipe = carry[:-1]
        v_n = nid_v[pl.ds((it + DEPTH) * L, L)]
        last_n = plsc.scan_count(v_n)[1]
        emit(jvec, pipe[0], pipe[1])
        return pipe[2:] + (v_n, last_n, jvec + L)

    prime = []
    for k in range(DEPTH):
        vk = nid_v[pl.ds(k * L, L)]
        prime += [vk, plsc.scan_count(vk)[1]]
    fin = lax.fori_loop(0, NIT - DEPTH, body, (*prime, lanes), unroll=16)
    j_f = fin[-1]
    for k in range(DEPTH):
        emit(j_f + k * L, fin[2 * k], fin[2 * k + 1])
    pltpu.sync_copy(slab_v, pos_hbm.at[pl.ds(wid * RANGE, RANGE)])


@functools.partial(
    pl.kernel,
    mesh=_mesh,
    compiler_params=pltpu.CompilerParams(
        needs_layout_passes=False, use_tc_tiling_on_sc=False),
    out_type=jax.ShapeDtypeStruct((BATCH, DIM), jnp.float32),
    scratch_types=[
        pltpu.VMEM((NCHUNK, CHUNK), jnp.int32),
        pltpu.VMEM((B_PER_W,), jnp.int32),
        pltpu.VMEM((B_PER_W, DIM), jnp.float32),
        pltpu.SemaphoreType.DMA,
        pltpu.SemaphoreType.DMA,
    ],
)
def _gather_out(nid_hbm, pos_hbm, x_hbm, out_hbm, nid_v, w_v, rows_v, s1, s2):
    wid = lax.axis_index("s")
    base = wid * B_PER_W
    pltpu.sync_copy(nid_hbm.at[pl.ds(wid * NCHUNK, NCHUNK)], nid_v)
    h1 = [pltpu.async_copy(pos_hbm.at[nid_v.at[c]],
                           w_v.at[pl.ds(c * CHUNK, CHUNK)], s1)
          for c in range(NCHUNK)]
    h2 = []
    for c in range(NCHUNK):
        h1[c].wait()
        h2.append(pltpu.async_copy(
            x_hbm.at[w_v.at[pl.ds(c * CHUNK, CHUNK)]],
            rows_v.at[pl.ds(c * CHUNK, CHUNK)], s2))
    h3 = []
    for c in range(NCHUNK):
        h2[c].wait()
        h3.append(pltpu.async_copy(
            rows_v.at[pl.ds(c * CHUNK, CHUNK)],
            out_hbm.at[pl.ds(base + c * CHUNK, CHUNK)], s1))
    for h in h3:
        h.wait()


def kernel(emb, x, n_id):
    del emb
    pos = _build_pos(n_id)
    nid2 = n_id.reshape(NW * NCHUNK, CHUNK)
    return _gather_out(nid2, pos, x)


# speedup vs baseline: 1.3015x; 1.0006x over previous
import functools

import jax
import jax.numpy as jnp
from jax import lax
from jax.experimental import pallas as pl
from jax.experimental.pallas import tpu as pltpu
from jax.experimental.pallas import tpu_sc as plsc

NUM_EMB = 1_000_000
DIM = 64
BATCH = 16384
NS, L = 16, 16
NW = NS
RANGE = 65536
POS_PAD = RANGE * NW
B_PER_W = BATCH // NW
CHUNK = 128
NCHUNK = B_PER_W // CHUNK
NIT = BATCH // L

_mesh = plsc.VectorSubcoreMesh(
    core_axis_name="c", subcore_axis_name="s", num_cores=1)


@functools.partial(
    pl.kernel,
    mesh=_mesh,
    compiler_params=pltpu.CompilerParams(needs_layout_passes=False),
    out_type=jax.ShapeDtypeStruct((POS_PAD,), jnp.int32),
    scratch_types=[
        pltpu.VMEM((BATCH,), jnp.int32),
        pltpu.VMEM((RANGE,), jnp.int32),
    ],
)
def _build_pos(nid_hbm, pos_hbm, nid_v, slab_v):
    wid = lax.axis_index("s")
    pltpu.sync_copy(nid_hbm, nid_v)
    lanes = lax.iota(jnp.int32, L)

    def emit(jvec, v, last):
        m = last & (lax.shift_right_arithmetic(v, 16) == wid)
        plsc.store_scatter(slab_v, [v & (RANGE - 1)], jvec, mask=m)

    DEPTH = 12

    def body(it, carry):
        jvec = carry[-1]
        pipe = carry[:-1]
        v_n = nid_v[pl.ds((it + DEPTH) * L, L)]
        last_n = plsc.scan_count(v_n)[1]
        emit(jvec, pipe[0], pipe[1])
        return pipe[2:] + (v_n, last_n, jvec + L)

    prime = []
    for k in range(DEPTH):
        vk = nid_v[pl.ds(k * L, L)]
        prime += [vk, plsc.scan_count(vk)[1]]
    fin = lax.fori_loop(0, NIT - DEPTH, body, (*prime, lanes), unroll=16)
    j_f = fin[-1]
    for k in range(DEPTH):
        emit(j_f + k * L, fin[2 * k], fin[2 * k + 1])
    pltpu.sync_copy(slab_v, pos_hbm.at[pl.ds(wid * RANGE, RANGE)])


@functools.partial(
    pl.kernel,
    mesh=_mesh,
    compiler_params=pltpu.CompilerParams(
        needs_layout_passes=False, use_tc_tiling_on_sc=False),
    out_type=jax.ShapeDtypeStruct((BATCH, DIM), jnp.float32),
    scratch_types=[
        pltpu.VMEM((NCHUNK, CHUNK), jnp.int32),
        pltpu.VMEM((B_PER_W,), jnp.int32),
        pltpu.VMEM((B_PER_W, DIM), jnp.float32),
        pltpu.SemaphoreType.DMA,
        pltpu.SemaphoreType.DMA,
    ],
)
def _gather_out(nid_hbm, pos_hbm, x_hbm, out_hbm, nid_v, w_v, rows_v, s1, s2):
    wid = lax.axis_index("s")
    base = wid * B_PER_W
    pltpu.sync_copy(nid_hbm.at[pl.ds(wid * NCHUNK, NCHUNK)], nid_v)
    h1 = [pltpu.async_copy(pos_hbm.at[nid_v.at[c]],
                           w_v.at[pl.ds(c * CHUNK, CHUNK)], s1)
          for c in range(NCHUNK)]
    h2 = []
    for c in range(NCHUNK):
        h1[c].wait()
        h2.append(pltpu.async_copy(
            x_hbm.at[w_v.at[pl.ds(c * CHUNK, CHUNK)]],
            rows_v.at[pl.ds(c * CHUNK, CHUNK)], s2))
    h3 = []
    for c in range(NCHUNK):
        h2[c].wait()
        h3.append(pltpu.async_copy(
            rows_v.at[pl.ds(c * CHUNK, CHUNK)],
            out_hbm.at[pl.ds(base + c * CHUNK, CHUNK)], s1))
    for h in h3:
        h.wait()


def kernel(emb, x, n_id):
    del emb
    pos = _build_pos(n_id)
    nid2 = n_id.reshape(NW * NCHUNK, CHUNK)
    return _gather_out(nid2, pos, x)
